# baseline (speedup 1.0000x reference)
import functools

import jax
import jax.numpy as jnp
from jax import lax
from jax.experimental import pallas as pl
from jax.experimental.pallas import tpu as pltpu
from jax.experimental.pallas import tpu_sc as plsc

_N_CODES = 8192
_EMBED = 256
_ROWS = 8192
_R_TILE = 1024
_C_CHUNK = 2048
_H_CHUNK = 256


def _argmin_body(z_ref, e_ref, idx_ref, loss_ref, esq_ref):
    i = pl.program_id(0)

    @pl.when(i == 0)
    def _esq():
        e = e_ref[...]
        esq_ref[...] = jnp.sum(e * e, axis=1).reshape(1, _N_CODES)

    zb = z_ref[...].reshape(_EMBED, _R_TILE)
    x = zb.T
    x_sq = jnp.sum(x * x, axis=1, keepdims=True)
    xb = x.astype(jnp.bfloat16)
    cols = lax.broadcasted_iota(jnp.int32, (_R_TILE, _C_CHUNK), 1)

    def tile_d(c):
        ec = e_ref[pl.ds(c * _C_CHUNK, _C_CHUNK), :]
        esqc = esq_ref[0, pl.ds(c * _C_CHUNK, _C_CHUNK)]
        return (x_sq
                - 2.0 * lax.dot_general(xb, ec.astype(jnp.bfloat16),
                                        (((1,), (1,)), ((), ())),
                                        preferred_element_type=jnp.float32)
                + esqc[None, :])

    def minarg(d, base):
        m = jnp.min(d, axis=1)
        a = jnp.min(jnp.where(d == m[:, None], cols + base, _N_CODES), axis=1)
        return m, a

    def comb(m1, a1, m2, a2):
        take2 = (m2 < m1) | ((m2 == m1) & (a2 < a1))
        return jnp.where(take2, m2, m1), jnp.where(take2, a2, a1)

    c0_m, c0_a = comb(*minarg(tile_d(0), 0), *minarg(tile_d(1), _C_CHUNK))
    c1_m, c1_a = comb(*minarg(tile_d(2), 2 * _C_CHUNK),
                      *minarg(tile_d(3), 3 * _C_CHUNK))

    acc = c0_m.astype(jnp.bfloat16).astype(jnp.float32)
    idx, val = c0_a, c0_m
    upd = (c1_m < acc) | ((c1_m == acc) & (c1_a < idx))
    idx = jnp.where(upd, c1_a, idx)
    val = jnp.where(upd, c1_m, val)

    idx_ref[0, 0, :] = idx

    @pl.when(i == 0)
    def _init():
        loss_ref[...] = jnp.zeros((1, 1), jnp.float32)
    loss_ref[...] += jnp.sum(val).reshape(1, 1)


def _argmin_call(z, embeddings):
    n_steps = _ROWS // _R_TILE
    return pl.pallas_call(
        _argmin_body,
        grid=(n_steps,),
        in_specs=[
            pl.BlockSpec((1, _EMBED, 32, 32), lambda i: (i, 0, 0, 0)),
            pl.BlockSpec((_N_CODES, _EMBED), lambda i: (0, 0)),
        ],
        out_specs=[
            pl.BlockSpec((1, 1, _R_TILE), lambda i: (i, 0, 0)),
            pl.BlockSpec((1, 1), lambda i: (0, 0)),
        ],
        out_shape=[
            jax.ShapeDtypeStruct((n_steps, 1, _R_TILE), jnp.int32),
            jax.ShapeDtypeStruct((1, 1), jnp.float32),
        ],
        scratch_shapes=[pltpu.VMEM((1, _N_CODES), jnp.float32)],
        compiler_params=pltpu.CompilerParams(
            dimension_semantics=("arbitrary",)),
    )(z, embeddings)


def _ppl_body(idx_ref, out_ref):
    idx = idx_ref[...].astype(jnp.int32)

    def body(c, ent):
        ids = (lax.broadcasted_iota(jnp.int32, (_H_CHUNK, _ROWS), 0)
               + c * _H_CHUNK)
        eq = (ids == idx).astype(jnp.float32)
        counts = jnp.sum(eq, axis=1)
        p = counts * (1.0 / _ROWS)
        return ent - jnp.sum(p * jnp.log(p + 1e-10))

    ent = lax.fori_loop(0, _N_CODES // _H_CHUNK, body, jnp.float32(0.0))
    out_ref[...] = jnp.exp(ent).reshape(1, 1)


def _ppl_call(idx_row):
    return pl.pallas_call(
        _ppl_body,
        out_shape=jax.ShapeDtypeStruct((1, 1), jnp.float32),
    )(idx_row)


_IDX_CHUNK = 128


@functools.cache
def _gather_kernel_factory():
    info = plsc.get_sparse_core_info()
    num_cores = info.num_cores
    nw = info.num_cores * info.num_subcores
    b_per_w = _ROWS // nw
    n_chunks = b_per_w // _IDX_CHUNK

    def _gather_body(table_hbm, idx_hbm, out_hbm, idx_v, rows_v, sem):
        wid = lax.axis_index("s") * num_cores + lax.axis_index("c")
        base = wid * b_per_w
        pltpu.sync_copy(idx_hbm.at[wid], idx_v)
        copies = []
        for j in range(n_chunks):
            copies.append(pltpu.async_copy(
                table_hbm.at[idx_v.at[j]],
                rows_v.at[pl.ds(j * _IDX_CHUNK, _IDX_CHUNK)],
                sem))
        for cp in copies:
            cp.wait()
        pltpu.sync_copy(rows_v, out_hbm.at[pl.ds(base, b_per_w)])

    mesh = plsc.VectorSubcoreMesh(core_axis_name="c", subcore_axis_name="s")
    k = pl.kernel(
        _gather_body,
        mesh=mesh,
        out_type=jax.ShapeDtypeStruct((_ROWS, _EMBED), jnp.float32),
        scratch_types=[
            pltpu.VMEM((n_chunks, _IDX_CHUNK), jnp.int32),
            pltpu.VMEM((b_per_w, _EMBED), jnp.float32),
            pltpu.SemaphoreType.DMA,
        ],
    )
    return k, nw, n_chunks


def _gather_call(embeddings, idx):
    k, nw, n_chunks = _gather_kernel_factory()
    idx3 = idx.reshape(nw, n_chunks, _IDX_CHUNK)
    return k(embeddings, idx3)


def kernel(z, embeddings):
    B, C, H, W = z.shape
    idx_blocks, loss_sum = _argmin_call(z, embeddings)
    idx = idx_blocks.reshape(_ROWS)

    emb_flat = _gather_call(embeddings, idx)
    perplexity = _ppl_call(idx.reshape(1, _ROWS))[0, 0]

    emb = jnp.transpose(emb_flat.reshape(B, H, W, C), (0, 3, 1, 2))
    commitment_loss = 0.25 * loss_sum[0, 0] / (B * C * H * W)
    encoding_indices_r = idx.reshape(B, H, W)
    return emb, encoding_indices_r, commitment_loss, perplexity


# speedup vs baseline: 1.2187x; 1.2187x over previous
import functools

import jax
import jax.numpy as jnp
from jax import lax
from jax.experimental import pallas as pl
from jax.experimental.pallas import tpu as pltpu
from jax.experimental.pallas import tpu_sc as plsc

_N_CODES = 8192
_EMBED = 256
_ROWS = 8192
_R_TILE = 1024
_C_CHUNK = 2048
_H_CHUNK = 256


def _argmin_body(x_ref, e_ref, idx_ref, loss_ref, esq_ref):
    i = pl.program_id(0)

    @pl.when(i == 0)
    def _esq():
        e = e_ref[...]
        esq_ref[...] = jnp.sum(e * e, axis=1).reshape(1, _N_CODES)

    x = x_ref[...]
    x_sq = jnp.sum(x * x, axis=1, keepdims=True)
    xb = x.astype(jnp.bfloat16)
    cols = lax.broadcasted_iota(jnp.int32, (_R_TILE, _C_CHUNK), 1)

    def tile_d(c):
        ec = e_ref[pl.ds(c * _C_CHUNK, _C_CHUNK), :]
        esqc = esq_ref[0, pl.ds(c * _C_CHUNK, _C_CHUNK)]
        return (x_sq
                - 2.0 * lax.dot_general(xb, ec.astype(jnp.bfloat16),
                                        (((1,), (1,)), ((), ())),
                                        preferred_element_type=jnp.float32)
                + esqc[None, :])

    def minarg(d, base):
        m = jnp.min(d, axis=1)
        a = jnp.min(jnp.where(d == m[:, None], cols + base, _N_CODES), axis=1)
        return m, a

    def comb(m1, a1, m2, a2):
        take2 = (m2 < m1) | ((m2 == m1) & (a2 < a1))
        return jnp.where(take2, m2, m1), jnp.where(take2, a2, a1)

    c0_m, c0_a = comb(*minarg(tile_d(0), 0), *minarg(tile_d(1), _C_CHUNK))
    c1_m, c1_a = comb(*minarg(tile_d(2), 2 * _C_CHUNK),
                      *minarg(tile_d(3), 3 * _C_CHUNK))

    acc = c0_m.astype(jnp.bfloat16).astype(jnp.float32)
    idx, val = c0_a, c0_m
    upd = (c1_m < acc) | ((c1_m == acc) & (c1_a < idx))
    idx = jnp.where(upd, c1_a, idx)
    val = jnp.where(upd, c1_m, val)

    idx_ref[0, 0, :] = idx

    @pl.when(i == 0)
    def _init():
        loss_ref[...] = jnp.zeros((1, 1), jnp.float32)
    loss_ref[...] += jnp.sum(val).reshape(1, 1)


def _argmin_call(flat, embeddings):
    n_steps = _ROWS // _R_TILE
    return pl.pallas_call(
        _argmin_body,
        grid=(n_steps,),
        in_specs=[
            pl.BlockSpec((_R_TILE, _EMBED), lambda i: (i, 0)),
            pl.BlockSpec((_N_CODES, _EMBED), lambda i: (0, 0)),
        ],
        out_specs=[
            pl.BlockSpec((1, 1, _R_TILE), lambda i: (i, 0, 0)),
            pl.BlockSpec((1, 1), lambda i: (0, 0)),
        ],
        out_shape=[
            jax.ShapeDtypeStruct((n_steps, 1, _R_TILE), jnp.int32),
            jax.ShapeDtypeStruct((1, 1), jnp.float32),
        ],
        scratch_shapes=[pltpu.VMEM((1, _N_CODES), jnp.float32)],
        compiler_params=pltpu.CompilerParams(
            dimension_semantics=("arbitrary",)),
    )(flat, embeddings)


def _ppl_body(idx_ref, out_ref):
    idx = idx_ref[...].astype(jnp.int32)

    def body(c, ent):
        ids = (lax.broadcasted_iota(jnp.int32, (_H_CHUNK, _ROWS), 0)
               + c * _H_CHUNK)
        eq = (ids == idx).astype(jnp.float32)
        counts = jnp.sum(eq, axis=1)
        p = counts * (1.0 / _ROWS)
        return ent - jnp.sum(p * jnp.log(p + 1e-10))

    ent = lax.fori_loop(0, _N_CODES // _H_CHUNK, body, jnp.float32(0.0))
    out_ref[...] = jnp.exp(ent).reshape(1, 1)


def _ppl_call(idx_row):
    return pl.pallas_call(
        _ppl_body,
        out_shape=jax.ShapeDtypeStruct((1, 1), jnp.float32),
    )(idx_row)


_IDX_CHUNK = 128


@functools.cache
def _gather_kernel_factory():
    info = plsc.get_sparse_core_info()
    num_cores = info.num_cores
    nw = info.num_cores * info.num_subcores
    b_per_w = _ROWS // nw
    n_chunks = b_per_w // _IDX_CHUNK

    def _gather_body(table_hbm, idx_hbm, out_hbm, idx_v, rows_v, sem):
        wid = lax.axis_index("s") * num_cores + lax.axis_index("c")
        base = wid * b_per_w
        pltpu.sync_copy(idx_hbm.at[wid], idx_v)
        copies = []
        for j in range(n_chunks):
            copies.append(pltpu.async_copy(
                table_hbm.at[idx_v.at[j]],
                rows_v.at[pl.ds(j * _IDX_CHUNK, _IDX_CHUNK)],
                sem))
        for cp in copies:
            cp.wait()
        pltpu.sync_copy(rows_v, out_hbm.at[pl.ds(base, b_per_w)])

    mesh = plsc.VectorSubcoreMesh(core_axis_name="c", subcore_axis_name="s")
    k = pl.kernel(
        _gather_body,
        mesh=mesh,
        out_type=jax.ShapeDtypeStruct((_ROWS, _EMBED), jnp.float32),
        scratch_types=[
            pltpu.VMEM((n_chunks, _IDX_CHUNK), jnp.int32),
            pltpu.VMEM((b_per_w, _EMBED), jnp.float32),
            pltpu.SemaphoreType.DMA,
        ],
    )
    return k, nw, n_chunks


def _gather_call(embeddings, idx):
    k, nw, n_chunks = _gather_kernel_factory()
    idx3 = idx.reshape(nw, n_chunks, _IDX_CHUNK)
    return k(embeddings, idx3)


def kernel(z, embeddings):
    B, C, H, W = z.shape
    flat = jnp.transpose(z, (0, 2, 3, 1)).reshape(_ROWS, _EMBED)

    idx_blocks, loss_sum = _argmin_call(flat, embeddings)
    idx = idx_blocks.reshape(_ROWS)

    emb_flat = _gather_call(embeddings, idx)
    perplexity = _ppl_call(idx.reshape(1, _ROWS))[0, 0]

    emb = jnp.transpose(emb_flat.reshape(B, H, W, C), (0, 3, 1, 2))
    commitment_loss = 0.25 * loss_sum[0, 0] / (B * C * H * W)
    encoding_indices_r = idx.reshape(B, H, W)
    return emb, encoding_indices_r, commitment_loss, perplexity
